# Initial kernel scaffold; baseline (speedup 1.0000x reference)
import jax
import jax.numpy as jnp
from jax.experimental import pallas as pl


def kernel(dct_in, emb, W_ih, W_hh, b_ih, b_hh, W_fc, b_fc):
    raise NotImplementedError("write your pallas kernel here")


# speedup vs baseline: 1.2539x; 1.2539x over previous
import functools

import jax
import jax.numpy as jnp
from jax import lax
from jax.experimental import pallas as pl
from jax.experimental.pallas import tpu as pltpu
from jax.experimental.pallas import tpu_sc as plsc


_GATHER_CHUNK = 128


@functools.lru_cache(maxsize=None)
def _make_sc_gather(V, D, TB):
    info = plsc.get_sparse_core_info()
    NC, NS = info.num_cores, info.num_subcores
    NW = NC * NS
    CB = _GATHER_CHUNK
    assert TB % (NW * CB) == 0
    rows_per_w = TB // NW
    nch = rows_per_w // CB
    mesh = plsc.VectorSubcoreMesh(core_axis_name="c", subcore_axis_name="s")

    @functools.partial(
        pl.kernel,
        mesh=mesh,
        out_type=jax.ShapeDtypeStruct((TB, D), jnp.float32),
        scratch_types=[
            pltpu.VMEM((nch, CB), jnp.int32),
            pltpu.VMEM((2, CB, D), jnp.float32),
            pltpu.SemaphoreType.DMA,
            pltpu.SemaphoreType.DMA,
        ],
        compiler_params=pltpu.CompilerParams(use_tc_tiling_on_sc=False),
    )
    def gather_k(emb_hbm, idx_hbm, out_hbm, idx_v, rows_v, sem0, sem1):
        wid = lax.axis_index("s") * NC + lax.axis_index("c")
        base_chunk = wid * nch
        pltpu.sync_copy(idx_hbm.at[wid], idx_v)

        def fire(j, buf, sem):
            pltpu.async_copy(emb_hbm.at[idx_v.at[j]], rows_v.at[buf], sem)

        def drain(j, buf, sem):
            pltpu.make_async_copy(emb_hbm.at[idx_v.at[j]], rows_v.at[buf], sem).wait()
            pltpu.sync_copy(rows_v.at[buf], out_hbm.at[pl.ds((base_chunk + j) * CB, CB)])

        fire(0, 0, sem0)

        def body(i, carry):
            j = 2 * i
            fire(j + 1, 1, sem1)
            drain(j, 0, sem0)
            fire(j + 2, 0, sem0)
            drain(j + 1, 1, sem1)
            return carry

        lax.fori_loop(0, nch // 2 - 1, body, 0)
        j = nch - 2
        fire(j + 1, 1, sem1)
        drain(j, 0, sem0)
        drain(j + 1, 1, sem1)

    return gather_k


def _lstm_body(x_ref, wih_ref, whh_ref, b_ref, wfc_ref, bfc_ref, out_ref,
               h_ref, c_ref, *, H, T):
    t = pl.program_id(1)

    @pl.when(t == 0)
    def _():
        h_ref[...] = jnp.zeros_like(h_ref)
        c_ref[...] = jnp.zeros_like(c_ref)

    x = x_ref[0]
    h = h_ref[...]
    gates = (
        jnp.dot(x, wih_ref[...], preferred_element_type=jnp.float32)
        + jnp.dot(h, whh_ref[...], preferred_element_type=jnp.float32)
        + b_ref[...]
    )
    i = jax.nn.sigmoid(gates[:, 0 * H:1 * H])
    f = jax.nn.sigmoid(gates[:, 1 * H:2 * H])
    g = jnp.tanh(gates[:, 2 * H:3 * H])
    o = jax.nn.sigmoid(gates[:, 3 * H:4 * H])
    c = f * c_ref[...] + i * g
    h = o * jnp.tanh(c)
    c_ref[...] = c
    h_ref[...] = h

    @pl.when(t == T - 1)
    def _():
        logits = (
            jnp.dot(h, wfc_ref[...], preferred_element_type=jnp.float32)
            + bfc_ref[...]
        )
        m = jnp.max(logits, axis=-1, keepdims=True)
        s = logits - m
        lse = jnp.log(jnp.sum(jnp.exp(s), axis=-1, keepdims=True))
        out_ref[...] = s - lse


def _lstm_call(x, wih_t, whh_t, b2, wfc_t, bfc2, *, bt=512):
    T, B, D = x.shape
    H = whh_t.shape[0]
    A = wfc_t.shape[1]
    grid = (B // bt, T)
    return pl.pallas_call(
        functools.partial(_lstm_body, H=H, T=T),
        grid=grid,
        in_specs=[
            pl.BlockSpec((1, bt, D), lambda b, t: (t, b, 0)),
            pl.BlockSpec((D, 4 * H), lambda b, t: (0, 0)),
            pl.BlockSpec((H, 4 * H), lambda b, t: (0, 0)),
            pl.BlockSpec((1, 4 * H), lambda b, t: (0, 0)),
            pl.BlockSpec((H, A), lambda b, t: (0, 0)),
            pl.BlockSpec((1, A), lambda b, t: (0, 0)),
        ],
        out_specs=pl.BlockSpec((bt, A), lambda b, t: (b, 0)),
        out_shape=jax.ShapeDtypeStruct((B, A), jnp.float32),
        scratch_shapes=[
            pltpu.VMEM((bt, H), jnp.float32),
            pltpu.VMEM((bt, H), jnp.float32),
        ],
        compiler_params=pltpu.CompilerParams(
            dimension_semantics=("parallel", "arbitrary"),
        ),
    )(x, wih_t, whh_t, b2, wfc_t, bfc2)


def kernel(dct_in, emb, W_ih, W_hh, b_ih, b_hh, W_fc, b_fc):
    B, T = dct_in.shape
    V, D = emb.shape
    H = W_hh.shape[1]
    A = W_fc.shape[0]

    idx = dct_in.T.reshape(-1).astype(jnp.int32)
    TB = T * B
    info = plsc.get_sparse_core_info()
    NW = info.num_cores * info.num_subcores
    CB = _GATHER_CHUNK
    x = _make_sc_gather(V, D, TB)(emb, idx.reshape(NW, TB // (NW * CB), CB))
    x = x.reshape(T, B, D)

    b2 = (b_ih + b_hh).reshape(1, 4 * H)
    out = _lstm_call(x, W_ih.T, W_hh.T, b2, W_fc.T, b_fc.reshape(1, A))
    return out


# speedup vs baseline: 1.3388x; 1.0677x over previous
import functools

import jax
import jax.numpy as jnp
from jax import lax
from jax.experimental import pallas as pl
from jax.experimental.pallas import tpu as pltpu
from jax.experimental.pallas import tpu_sc as plsc


_CB = 128


@functools.lru_cache(maxsize=None)
def _make_sc_gather(V, D, TB):
    info = plsc.get_sparse_core_info()
    NC, NS = info.num_cores, info.num_subcores
    NW = NC * NS
    CB = _CB
    assert TB % (NW * CB) == 0
    rows_per_w = TB // NW
    nch = rows_per_w // CB
    mesh = plsc.VectorSubcoreMesh(core_axis_name="c", subcore_axis_name="s")

    PC = CB // 2
    pairs_per_w = rows_per_w // 2

    @functools.partial(
        pl.kernel,
        mesh=mesh,
        out_type=jax.ShapeDtypeStruct((TB // 2, 128), jnp.float32),
        scratch_types=[
            pltpu.VMEM((pairs_per_w,), jnp.int32),
            pltpu.VMEM((pairs_per_w,), jnp.int32),
            pltpu.VMEM((2, 2, PC, 64), jnp.float32),
            pltpu.SemaphoreType.DMA,
            pltpu.SemaphoreType.DMA,
        ],
        compiler_params=pltpu.CompilerParams(use_tc_tiling_on_sc=False),
    )
    def gather_k(emb_hbm, idxe_hbm, idxo_hbm, out_hbm, idxe_v, idxo_v,
                 rows_v, sem0, sem1):
        wid = lax.axis_index("s") * NC + lax.axis_index("c")
        pltpu.sync_copy(idxe_hbm.at[pl.ds(wid * pairs_per_w, pairs_per_w)], idxe_v)
        pltpu.sync_copy(idxo_hbm.at[pl.ds(wid * pairs_per_w, pairs_per_w)], idxo_v)

        def fire(j, buf, sem):
            pltpu.async_copy(
                emb_hbm.at[idxe_v.at[pl.ds(j * PC, PC)]],
                rows_v.at[buf, 0],
                sem,
            )
            pltpu.async_copy(
                emb_hbm.at[idxo_v.at[pl.ds(j * PC, PC)]],
                rows_v.at[buf, 1],
                sem,
            )

        def drain(j, buf, sem):
            pltpu.make_async_copy(
                emb_hbm.at[idxe_v.at[pl.ds(j * PC, PC)]],
                rows_v.at[buf, 0],
                sem,
            ).wait()
            pltpu.make_async_copy(
                emb_hbm.at[idxo_v.at[pl.ds(j * PC, PC)]],
                rows_v.at[buf, 1],
                sem,
            ).wait()
            row0 = wid * pairs_per_w + j * PC
            pltpu.sync_copy(rows_v.at[buf, 0],
                            out_hbm.at[pl.ds(row0, PC), pl.ds(0, 64)])
            pltpu.sync_copy(rows_v.at[buf, 1],
                            out_hbm.at[pl.ds(row0, PC), pl.ds(64, 64)])

        fire(0, 0, sem0)

        def body(i, carry):
            j = 2 * i
            fire(j + 1, 1, sem1)
            drain(j, 0, sem0)
            fire(j + 2, 0, sem0)
            drain(j + 1, 1, sem1)
            return carry

        lax.fori_loop(0, nch // 2 - 1, body, 0)
        j = nch - 2
        fire(j + 1, 1, sem1)
        drain(j, 0, sem0)
        drain(j + 1, 1, sem1)

    return gather_k


def _lstm_body(x_ref, wih_ref, whh_ref, b_ref, wfc_ref, bfc_ref, out_ref,
               h_ref, c_ref, *, H, T, A, D):
    t = pl.program_id(1)

    @pl.when(t == 0)
    def _():
        h_ref[...] = jnp.zeros_like(h_ref)
        c_ref[...] = jnp.zeros_like(c_ref)

    x = x_ref[0]
    for half in (0, 1):
        g = (
            jnp.dot(x[:, half * D:(half + 1) * D], wih_ref[...],
                    preferred_element_type=jnp.float32)
            + jnp.dot(h_ref[:, half * H:(half + 1) * H], whh_ref[...],
                      preferred_element_type=jnp.float32)
            + b_ref[...]
        )
        i_g = jax.nn.sigmoid(g[:, 0 * H:1 * H])
        f_g = jax.nn.sigmoid(g[:, 1 * H:2 * H])
        g_g = jnp.tanh(g[:, 2 * H:3 * H])
        o_g = jax.nn.sigmoid(g[:, 3 * H:4 * H])
        c_new = f_g * c_ref[:, half * H:(half + 1) * H] + i_g * g_g
        h_new = o_g * jnp.tanh(c_new)
        c_ref[:, half * H:(half + 1) * H] = c_new
        h_ref[:, half * H:(half + 1) * H] = h_new

    @pl.when(t == T - 1)
    def _():
        for half in (0, 1):
            logits = (
                jnp.dot(h_ref[:, half * H:(half + 1) * H], wfc_ref[...],
                        preferred_element_type=jnp.float32)
                + bfc_ref[...]
            )
            m = jnp.max(logits, axis=-1, keepdims=True)
            s = logits - m
            lse = jnp.log(jnp.sum(jnp.exp(s), axis=-1, keepdims=True))
            out_ref[:, half * A:(half + 1) * A] = s - lse


def _lstm_call(x, wih, whh, b1, wfc, bfc1, *, bth=256):
    T, BH, TWOD = x.shape
    D = TWOD // 2
    H = whh.shape[0]
    A = wfc.shape[1]
    grid = (BH // bth, T)
    return pl.pallas_call(
        functools.partial(_lstm_body, H=H, T=T, A=A, D=D),
        grid=grid,
        in_specs=[
            pl.BlockSpec((1, bth, 2 * D), lambda b, t: (t, b, 0)),
            pl.BlockSpec((D, 4 * H), lambda b, t: (0, 0)),
            pl.BlockSpec((H, 4 * H), lambda b, t: (0, 0)),
            pl.BlockSpec((1, 4 * H), lambda b, t: (0, 0)),
            pl.BlockSpec((H, A), lambda b, t: (0, 0)),
            pl.BlockSpec((1, A), lambda b, t: (0, 0)),
        ],
        out_specs=pl.BlockSpec((bth, 2 * A), lambda b, t: (b, 0)),
        out_shape=jax.ShapeDtypeStruct((BH, 2 * A), jnp.float32),
        scratch_shapes=[
            pltpu.VMEM((bth, 2 * H), jnp.float32),
            pltpu.VMEM((bth, 2 * H), jnp.float32),
        ],
        compiler_params=pltpu.CompilerParams(
            dimension_semantics=("parallel", "arbitrary"),
        ),
    )(x, wih, whh, b1, wfc, bfc1)


def kernel(dct_in, emb, W_ih, W_hh, b_ih, b_hh, W_fc, b_fc):
    B, T = dct_in.shape
    V, D = emb.shape
    H = W_hh.shape[1]
    A = W_fc.shape[0]
    TB = T * B

    idx = dct_in.T.reshape(TB // 2, 2).astype(jnp.int32)
    x2 = _make_sc_gather(V, D, TB)(emb, idx[:, 0], idx[:, 1])
    x3 = x2.reshape(T, B // 2, 2 * D)

    b1 = (b_ih + b_hh).reshape(1, 4 * H)
    bfc1 = b_fc.reshape(1, A)

    outp = _lstm_call(x3, W_ih.T, W_hh.T, b1, W_fc.T, bfc1)
    return outp.reshape(B, A)


# speedup vs baseline: 1.7565x; 1.3120x over previous
import functools

import jax
import jax.numpy as jnp
from jax import lax
from jax.experimental import pallas as pl
from jax.experimental.pallas import tpu as pltpu
from jax.experimental.pallas import tpu_sc as plsc


def _expand_body(embt_ref, out_ref, *, D, C):
    x = embt_ref[...]
    xt = jnp.transpose(x, (1, 0))
    out_ref[...] = jnp.concatenate(
        [xt, jnp.zeros((C, 128 - D), jnp.float32)], axis=1
    )


def _expand_call(embt, *, C=4096):
    D, V = embt.shape
    return pl.pallas_call(
        functools.partial(_expand_body, D=D, C=C),
        grid=(pl.cdiv(V, C),),
        in_specs=[pl.BlockSpec((D, C), lambda i: (0, i))],
        out_specs=pl.BlockSpec((C, 128), lambda i: (i, 0)),
        out_shape=jax.ShapeDtypeStruct((V, 128), jnp.float32),
        compiler_params=pltpu.CompilerParams(
            dimension_semantics=("arbitrary",),
        ),
    )(embt)


@functools.lru_cache(maxsize=None)
def _make_sc_gather(V, B, T):
    info = plsc.get_sparse_core_info()
    NC, NS = info.num_cores, info.num_subcores
    NW = NC * NS
    BB = B // NW
    assert BB == 128
    mesh = plsc.VectorSubcoreMesh(core_axis_name="c", subcore_axis_name="s")

    @functools.partial(
        pl.kernel,
        mesh=mesh,
        out_type=jax.ShapeDtypeStruct((T, B, 128), jnp.float32),
        scratch_types=[
            pltpu.VMEM((T, BB), jnp.int32),
            pltpu.VMEM((2, BB, 128), jnp.float32),
            pltpu.SemaphoreType.DMA,
            pltpu.SemaphoreType.DMA,
        ],
        compiler_params=pltpu.CompilerParams(use_tc_tiling_on_sc=False),
    )
    def gather_k(embp_hbm, idxt_hbm, out_hbm, idx_v, rows_v, sem0, sem1):
        wid = lax.axis_index("s") * NC + lax.axis_index("c")
        b0 = wid * BB
        pltpu.sync_copy(idxt_hbm.at[:, pl.ds(b0, BB)], idx_v)

        def fire(t, buf, sem):
            pltpu.async_copy(embp_hbm.at[idx_v.at[t]], rows_v.at[buf], sem)

        def drain(t, buf, sem):
            pltpu.make_async_copy(
                embp_hbm.at[idx_v.at[t]], rows_v.at[buf], sem
            ).wait()
            pltpu.sync_copy(rows_v.at[buf], out_hbm.at[t, pl.ds(b0, BB)])

        fire(0, 0, sem0)

        def body(i, carry):
            t = 2 * i
            fire(t + 1, 1, sem1)
            drain(t, 0, sem0)
            fire(t + 2, 0, sem0)
            drain(t + 1, 1, sem1)
            return carry

        lax.fori_loop(0, T // 2 - 1, body, 0)
        t = T - 2
        fire(t + 1, 1, sem1)
        drain(t, 0, sem0)
        drain(t + 1, 1, sem1)

    return gather_k


def _lstm_body(x_ref, wih_ref, whh_ref, b_ref, wfc_ref, bfc_ref, out_ref,
               h_ref, c_ref, *, H, T, D):
    t = pl.program_id(1)

    @pl.when(t == 0)
    def _():
        h_ref[...] = jnp.zeros_like(h_ref)
        c_ref[...] = jnp.zeros_like(c_ref)

    x = x_ref[0][:, 0:D]
    h = h_ref[...]
    g = (
        jnp.dot(x, wih_ref[...], preferred_element_type=jnp.float32)
        + jnp.dot(h, whh_ref[...], preferred_element_type=jnp.float32)
        + b_ref[...]
    )
    i_g = jax.nn.sigmoid(g[:, 0 * H:1 * H])
    f_g = jax.nn.sigmoid(g[:, 1 * H:2 * H])
    g_g = jnp.tanh(g[:, 2 * H:3 * H])
    o_g = jax.nn.sigmoid(g[:, 3 * H:4 * H])
    c_new = f_g * c_ref[...] + i_g * g_g
    h_new = o_g * jnp.tanh(c_new)
    c_ref[...] = c_new
    h_ref[...] = h_new

    @pl.when(t == T - 1)
    def _():
        logits = (
            jnp.dot(h_new, wfc_ref[...], preferred_element_type=jnp.float32)
            + bfc_ref[...]
        )
        m = jnp.max(logits, axis=-1, keepdims=True)
        s = logits - m
        lse = jnp.log(jnp.sum(jnp.exp(s), axis=-1, keepdims=True))
        out_ref[...] = s - lse


def _lstm_call(x, wih, whh, b1, wfc, bfc1, *, bt=512):
    T, B, _ = x.shape
    D = wih.shape[0]
    H = whh.shape[0]
    A = wfc.shape[1]
    grid = (B // bt, T)
    return pl.pallas_call(
        functools.partial(_lstm_body, H=H, T=T, D=D),
        grid=grid,
        in_specs=[
            pl.BlockSpec((1, bt, 128), lambda b, t: (t, b, 0)),
            pl.BlockSpec((D, 4 * H), lambda b, t: (0, 0)),
            pl.BlockSpec((H, 4 * H), lambda b, t: (0, 0)),
            pl.BlockSpec((1, 4 * H), lambda b, t: (0, 0)),
            pl.BlockSpec((H, A), lambda b, t: (0, 0)),
            pl.BlockSpec((1, A), lambda b, t: (0, 0)),
        ],
        out_specs=pl.BlockSpec((bt, A), lambda b, t: (b, 0)),
        out_shape=jax.ShapeDtypeStruct((B, A), jnp.float32),
        scratch_shapes=[
            pltpu.VMEM((bt, H), jnp.float32),
            pltpu.VMEM((bt, H), jnp.float32),
        ],
        compiler_params=pltpu.CompilerParams(
            dimension_semantics=("parallel", "arbitrary"),
        ),
    )(x, wih, whh, b1, wfc, bfc1)


def kernel(dct_in, emb, W_ih, W_hh, b_ih, b_hh, W_fc, b_fc):
    B, T = dct_in.shape
    V, D = emb.shape
    H = W_hh.shape[1]
    A = W_fc.shape[0]

    embp = _expand_call(emb.T)
    idxt = dct_in.T.astype(jnp.int32)
    x3 = _make_sc_gather(V, B, T)(embp, idxt)

    b1 = (b_ih + b_hh).reshape(1, 4 * H)
    bfc1 = b_fc.reshape(1, A)
    return _lstm_call(x3, W_ih.T, W_hh.T, b1, W_fc.T, bfc1)
